# baseline (speedup 1.0000x reference)
import jax
import jax.numpy as jnp
from jax import lax
from jax.experimental import pallas as pl
from jax.experimental.pallas import tpu as pltpu
from jax.experimental.pallas import tpu_sc as plsc

N, D, H, C, T, B, E, HS, K, NL = 10000, 768, 256, 14, 128, 16, 160000, 128, 8, 3
ALPHA = 0.1
HH = H // 2
NSUB = 16
WIN = 64
NBUF = 4
EPS = 10240
EPAD = EPS * NSUB
NWIN_SUB = EPS // WIN
CHUNK = 40
NCHUNK = NWIN_SUB // CHUNK
NACC = 10112
ROWS_PER_SUB = NACC // NSUB
TBF = T * B
G3 = 3 * HS


def _sc_scatter_body(src_hbm, dst_hbm, g0_hbm, g1_hbm, zeros_hbm,
                     acc0_hbm, acc1_hbm,
                     srcw, dstw, rows0, rows1, rows2, rows3, acc_sh,
                     gsem0, gsem1, gsem2, gsem3,
                     ssem0, ssem1, ssem2, ssem3):
    core = lax.axis_index("c")
    sub = lax.axis_index("s")
    zbase = sub * ROWS_PER_SUB
    pltpu.sync_copy(zeros_hbm, acc_sh.at[pl.ds(zbase, ROWS_PER_SUB)])
    plsc.subcore_barrier()

    def do_half(g_hbm, acc_out):
        base_row = sub * NWIN_SUB
        bufs = (rows0, rows1, rows2, rows3)
        gsems = (gsem0, gsem1, gsem2, gsem3)
        ssems = (ssem0, ssem1, ssem2, ssem3)

        def fire(w, b):
            pltpu.async_copy(g_hbm.at[srcw.at[w]], bufs[b], gsems[b])

        def wait_gather(w, b):
            pltpu.make_async_copy(g_hbm.at[srcw.at[w]], bufs[b],
                                  gsems[b]).wait()

        def scatter(w, b):
            pltpu.async_copy(bufs[b], acc_sh.at[dstw.at[w]], ssems[b],
                             add=True)

        def wait_scatter(w, b):
            pltpu.make_async_copy(bufs[b], acc_sh.at[dstw.at[w]],
                                  ssems[b]).wait()

        @pl.loop(0, NCHUNK)
        def _(c):
            crow = base_row + c * CHUNK
            pltpu.sync_copy(src_hbm.at[pl.ds(crow, CHUNK)], srcw)
            pltpu.sync_copy(dst_hbm.at[pl.ds(crow, CHUNK)], dstw)
            fire(0, 0)
            fire(1, 1)
            for w in (0, 1):
                wait_gather(w, w)
                scatter(w, w)
                fire(w + 2, w + 2)

            @pl.loop(0, (CHUNK - 4) // NBUF)
            def _(i):
                for j in range(NBUF):
                    w = 2 + NBUF * i + j
                    b = (2 + j) % NBUF
                    b2 = j % NBUF
                    wait_gather(w, b)
                    scatter(w, b)
                    wait_scatter(w - 2, b2)
                    fire(w + 2, b2)

            for j in range(2):
                w = CHUNK - 2 + j
                b = (2 + j) % NBUF
                wait_gather(w, b)
                scatter(w, b)
            for j in range(NBUF):
                w = CHUNK - NBUF + j
                wait_scatter(w, w % NBUF)

        plsc.subcore_barrier()
        obase = sub * ROWS_PER_SUB
        pltpu.sync_copy(acc_sh.at[pl.ds(obase, ROWS_PER_SUB)],
                        acc_out.at[pl.ds(obase, ROWS_PER_SUB)])

    @pl.when(core == 0)
    def _():
        do_half(g0_hbm, acc0_hbm)

    @pl.when(core == 1)
    def _():
        do_half(g1_hbm, acc1_hbm)


def _sc_scatter_pass(src2d, dst2d, g0, g1, zeros_init):
    mesh = plsc.VectorSubcoreMesh(core_axis_name="c", subcore_axis_name="s")
    acc_ty = jax.ShapeDtypeStruct((NACC, HH), jnp.float32)
    kern = pl.kernel(
        _sc_scatter_body,
        out_type=[acc_ty, acc_ty],
        mesh=mesh,
        scratch_types=[
            pltpu.VMEM((CHUNK, WIN), jnp.int32),
            pltpu.VMEM((CHUNK, WIN), jnp.int32),
            pltpu.VMEM((WIN, HH), jnp.float32),
            pltpu.VMEM((WIN, HH), jnp.float32),
            pltpu.VMEM((WIN, HH), jnp.float32),
            pltpu.VMEM((WIN, HH), jnp.float32),
            pltpu.VMEM_SHARED((NACC, HH), jnp.float32),
            pltpu.SemaphoreType.DMA,
            pltpu.SemaphoreType.DMA,
            pltpu.SemaphoreType.DMA,
            pltpu.SemaphoreType.DMA,
            pltpu.SemaphoreType.DMA,
            pltpu.SemaphoreType.DMA,
            pltpu.SemaphoreType.DMA,
            pltpu.SemaphoreType.DMA,
        ],
    )
    return kern(src2d, dst2d, g0, g1, zeros_init)


MBLK = 1000


def _proj_body(x_ref, w_ref, b_ref, o_ref):
    o_ref[...] = jnp.maximum(
        jnp.dot(x_ref[...], w_ref[...], preferred_element_type=jnp.float32)
        + b_ref[...], 0.0)


def _proj(x, w, b):
    return pl.pallas_call(
        _proj_body,
        grid=(N // MBLK,),
        in_specs=[
            pl.BlockSpec((MBLK, D), lambda i: (i, 0)),
            pl.BlockSpec((D, H), lambda i: (0, 0)),
            pl.BlockSpec((1, H), lambda i: (0, 0)),
        ],
        out_specs=pl.BlockSpec((MBLK, H), lambda i: (i, 0)),
        out_shape=jax.ShapeDtypeStruct((N, H), jnp.float32),
    )(x, w, b)


def _mm_scale_body(h_ref, w_ref, ideg_ref, g0_ref, g1_ref):
    ht = jnp.dot(h_ref[...], w_ref[...], preferred_element_type=jnp.float32)
    dinv = lax.rsqrt(ideg_ref[...] + 1.0)
    g = ht * dinv
    g0_ref[...] = g[:, :HH]
    g1_ref[...] = g[:, HH:]


def _mm_scale(h, w, ideg):
    half = jax.ShapeDtypeStruct((N, HH), jnp.float32)
    return pl.pallas_call(
        _mm_scale_body,
        grid=(N // MBLK,),
        in_specs=[
            pl.BlockSpec((MBLK, H), lambda i: (i, 0)),
            pl.BlockSpec((H, H), lambda i: (0, 0)),
            pl.BlockSpec((MBLK, 1), lambda i: (i, 0)),
        ],
        out_specs=[pl.BlockSpec((MBLK, HH), lambda i: (i, 0))] * 2,
        out_shape=[half, half],
    )(h, w, ideg)


def _blend_body(a0_ref, a1_ref, g0_ref, g1_ref, hi_ref, ideg_ref,
                hn0_ref, hn1_ref, gn0_ref, gn1_ref):
    dinv = lax.rsqrt(ideg_ref[...] + 1.0)
    sc = (1.0 - ALPHA) * dinv
    h0 = (a0_ref[...] + g0_ref[...]) * sc + ALPHA * hi_ref[:, :HH]
    h1 = (a1_ref[...] + g1_ref[...]) * sc + ALPHA * hi_ref[:, HH:]
    hn0_ref[...] = h0
    hn1_ref[...] = h1
    gn0_ref[...] = h0 * dinv
    gn1_ref[...] = h1 * dinv


def _blend(a0, a1, g0, g1, h_init, ideg):
    half = jax.ShapeDtypeStruct((N, HH), jnp.float32)
    return pl.pallas_call(
        _blend_body,
        grid=(N // MBLK,),
        in_specs=[pl.BlockSpec((MBLK, HH), lambda i: (i, 0))] * 4 + [
            pl.BlockSpec((MBLK, H), lambda i: (i, 0)),
            pl.BlockSpec((MBLK, 1), lambda i: (i, 0)),
        ],
        out_specs=[pl.BlockSpec((MBLK, HH), lambda i: (i, 0))] * 4,
        out_shape=[half, half, half, half],
    )(a0, a1, g0, g1, h_init, ideg)


def _epilogue_body(h_ref, ew_ref, lng_ref, lnb_ref,
                   p00, p01, p10, p11, p20, p21, p30, p31, out_ref):
    ew = ew_ref[...]
    halves = ((p00, p01), (p10, p11), (p20, p21), (p30, p31))
    acc = jnp.zeros((MBLK, H), jnp.float32)
    for t in range(4):
        hp = jnp.concatenate([halves[t][0][...], halves[t][1][...]], axis=1)
        acc = acc + ew[0:1, t:t + 1] * hp
    m = jnp.mean(acc, axis=1, keepdims=True)
    v = jnp.mean((acc - m) ** 2, axis=1, keepdims=True)
    y = (acc - m) * lax.rsqrt(v + 1e-5) * lng_ref[...] + lnb_ref[...]
    g = y * 0.5 * (1.0 + lax.erf(y * jnp.float32(0.7071067811865476)))
    out_ref[...] = g + h_ref[...]


def _epilogue(h, ew, lng, lnb, hp_halves):
    flat = [a for pair in hp_halves for a in pair]
    return pl.pallas_call(
        _epilogue_body,
        grid=(N // MBLK,),
        in_specs=[
            pl.BlockSpec((MBLK, H), lambda i: (i, 0)),
            pl.BlockSpec((1, 4), lambda i: (0, 0)),
            pl.BlockSpec((1, H), lambda i: (0, 0)),
            pl.BlockSpec((1, H), lambda i: (0, 0)),
        ] + [pl.BlockSpec((MBLK, HH), lambda i: (i, 0))] * 8,
        out_specs=pl.BlockSpec((MBLK, H), lambda i: (i, 0)),
        out_shape=jax.ShapeDtypeStruct((N, H), jnp.float32),
    )(h, ew, lng, lnb, *flat)


def _mm_bias_body(x_ref, w_ref, b_ref, o_ref):
    o_ref[...] = jnp.dot(x_ref[...], w_ref[...],
                         preferred_element_type=jnp.float32) + b_ref[...]


def _mm_bias(x, w, b, rblk):
    rows, kdim = x.shape
    ndim = w.shape[1]
    return pl.pallas_call(
        _mm_bias_body,
        grid=(rows // rblk,),
        in_specs=[
            pl.BlockSpec((rblk, kdim), lambda i: (i, 0)),
            pl.BlockSpec((kdim, ndim), lambda i: (0, 0)),
            pl.BlockSpec((1, ndim), lambda i: (0, 0)),
        ],
        out_specs=pl.BlockSpec((rblk, ndim), lambda i: (i, 0)),
        out_shape=jax.ShapeDtypeStruct((rows, ndim), jnp.float32),
    )(x, w, b)


def _gru_cell(gi, gh, bn, h):
    r = jax.nn.sigmoid(gi[:, :HS] + gh[:, :HS])
    z = jax.nn.sigmoid(gi[:, HS:2 * HS] + gh[:, HS:2 * HS])
    nn_ = jnp.tanh(gi[:, 2 * HS:] + r * (gh[:, 2 * HS:] + bn))
    return (1.0 - z) * nn_ + z * h


def _scan_mid_body(gif_ref, gib_ref, uf_ref, ub_ref, bnf_ref, bnb_ref,
                   y_ref):
    uf = uf_ref[...]
    ub = ub_ref[...]
    bnf = bnf_ref[...]
    bnb = bnb_ref[...]

    def step(t, carry):
        hf, hb = carry
        tb = T - 1 - t
        ghf = jnp.dot(hf, uf, preferred_element_type=jnp.float32)
        ghb = jnp.dot(hb, ub, preferred_element_type=jnp.float32)
        gf = gif_ref[pl.ds(t * B, B), :]
        gb = gib_ref[pl.ds(tb * B, B), :]
        hf2 = _gru_cell(gf, ghf, bnf, hf)
        hb2 = _gru_cell(gb, ghb, bnb, hb)
        y_ref[pl.ds(t * B, B), :HS] = hf2
        y_ref[pl.ds(tb * B, B), HS:] = hb2
        return (hf2, hb2)

    h0 = jnp.zeros((B, HS), jnp.float32)
    lax.fori_loop(0, T, step, (h0, h0))


def _scan_mid(gif, gib, uf, ub, bnf, bnb):
    return pl.pallas_call(
        _scan_mid_body,
        out_shape=jax.ShapeDtypeStruct((TBF, 2 * HS), jnp.float32),
    )(gif, gib, uf, ub, bnf, bnb)


def _scan_final_body(gif_ref, gib_ref, uf_ref, ub_ref, bnf_ref, bnb_ref,
                     w1_ref, b1_ref, w2_ref, b2_ref, out_ref):
    uf = uf_ref[...]
    ub = ub_ref[...]
    bnf = bnf_ref[...]
    bnb = bnb_ref[...]
    neg = jnp.float32(-jnp.inf)

    def step(t, carry):
        hf, hb, sf, sb, mf, mb = carry
        tb = T - 1 - t
        ghf = jnp.dot(hf, uf, preferred_element_type=jnp.float32)
        ghb = jnp.dot(hb, ub, preferred_element_type=jnp.float32)
        gf = gif_ref[pl.ds(t * B, B), :]
        gb = gib_ref[pl.ds(tb * B, B), :]
        hf2 = _gru_cell(gf, ghf, bnf, hf)
        hb2 = _gru_cell(gb, ghb, bnb, hb)
        return (hf2, hb2, sf + hf2, sb + hb2,
                jnp.maximum(mf, hf2), jnp.maximum(mb, hb2))

    h0 = jnp.zeros((B, HS), jnp.float32)
    m0 = jnp.full((B, HS), neg, jnp.float32)
    _, _, sf, sb, mf, mb = lax.fori_loop(0, T, step, (h0, h0, h0, h0, m0, m0))
    sp = jnp.concatenate([sf / T + mf, sb / T + mb], axis=1)
    y = jnp.maximum(jnp.dot(sp, w1_ref[...],
                            preferred_element_type=jnp.float32)
                    + b1_ref[...], 0.0)
    out_ref[...] = jnp.dot(y, w2_ref[...],
                           preferred_element_type=jnp.float32) + b2_ref[...]


def _scan_final(gif, gib, uf, ub, bnf, bnb, w1, b1, w2, b2):
    return pl.pallas_call(
        _scan_final_body,
        out_shape=jax.ShapeDtypeStruct((B, C), jnp.float32),
    )(gif, gib, uf, ub, bnf, bnb, w1, b1, w2, b2)


def _pool_body(off_ref, h_ref, w1_ref, b1_ref, w2_ref, b2_ref, sout_ref,
               out_ref):
    for b in range(B):
        lo = off_ref[b]
        hi = off_ref[b + 1]

        def step(i, carry):
            s, m = carry
            row = h_ref[pl.ds(lo + i, 1), :]
            return (s + row, jnp.maximum(m, row))

        s0 = jnp.zeros((1, H), jnp.float32)
        m0 = jnp.full((1, H), -jnp.inf, jnp.float32)
        s, m = lax.fori_loop(0, hi - lo, step, (s0, m0))
        cnt = jnp.maximum((hi - lo).astype(jnp.float32), 1.0)
        gfeat = s / cnt + m
        y = jnp.maximum(jnp.dot(gfeat, w1_ref[...],
                                preferred_element_type=jnp.float32)
                        + b1_ref[...], 0.0)
        go = jnp.dot(y, w2_ref[...],
                     preferred_element_type=jnp.float32) + b2_ref[...]
        out_ref[pl.ds(b, 1), :] = go + sout_ref[pl.ds(b, 1), :]


def _pool(offsets, h, w1, b1, w2, b2, sout):
    return pl.pallas_call(
        _pool_body,
        in_specs=[pl.BlockSpec(memory_space=pltpu.SMEM)] + [
            pl.BlockSpec((N, H), lambda: (0, 0)),
            pl.BlockSpec((H, HH), lambda: (0, 0)),
            pl.BlockSpec((1, HH), lambda: (0, 0)),
            pl.BlockSpec((HH, C), lambda: (0, 0)),
            pl.BlockSpec((1, C), lambda: (0, 0)),
            pl.BlockSpec((B, C), lambda: (0, 0)),
        ],
        out_specs=pl.BlockSpec((B, C), lambda: (0, 0)),
        out_shape=jax.ShapeDtypeStruct((B, C), jnp.float32),
    )(offsets, h, w1, b1, w2, b2, sout)


def kernel(x, edge_ast, edge_cfg, edge_dfg, edge_cdg, batch,
           sequence_features, proj_w, proj_b, edge_w, edge_imp, ln_g, ln_b,
           gru_wih_0f, gru_whh_0f, gru_bih_0f, gru_bhh_0f,
           gru_wih_0b, gru_whh_0b, gru_bih_0b, gru_bhh_0b,
           gru_wih_1f, gru_whh_1f, gru_bih_1f, gru_bhh_1f,
           gru_wih_1b, gru_whh_1b, gru_bih_1b, gru_bhh_1b,
           gmlp_w1, gmlp_b1, gmlp_w2, gmlp_b2,
           smlp_w1, smlp_b1, smlp_w2, smlp_b2):
    f32 = jnp.float32
    edges = (edge_ast, edge_cfg, edge_dfg, edge_cdg)

    npad = EPAD - E
    src2d, dst2d = [], []
    for e in edges:
        e = e.astype(jnp.int32)
        s = jnp.concatenate([e[0], jnp.zeros((npad,), jnp.int32)])
        d = jnp.concatenate([e[1], jnp.full((npad,), N, jnp.int32)])
        src2d.append(s.reshape(EPAD // WIN, WIN))
        dst2d.append(d.reshape(EPAD // WIN, WIN))

    zeros_init = jnp.zeros((ROWS_PER_SUB, HH), f32)
    ones_tab = jnp.ones((N, HH), f32)

    ideg = []
    for t in range(4):
        a0, _ = _sc_scatter_pass(src2d[t], dst2d[t], ones_tab, ones_tab,
                                 zeros_init)
        ideg.append(a0[:N, 0:1])

    xs_flat = jnp.swapaxes(sequence_features, 0, 1).reshape(TBF, D)

    def gate_prep(wih, whh, bih, bhh):
        gb = bih + jnp.concatenate([bhh[:2 * HS], jnp.zeros((HS,), f32)])
        return (wih.T, whh.T, gb.reshape(1, G3),
                bhh[2 * HS:].reshape(1, HS))

    w0f, u0f, gb0f, bn0f = gate_prep(gru_wih_0f, gru_whh_0f,
                                     gru_bih_0f, gru_bhh_0f)
    w0b, u0b, gb0b, bn0b = gate_prep(gru_wih_0b, gru_whh_0b,
                                     gru_bih_0b, gru_bhh_0b)
    w1f, u1f, gb1f, bn1f = gate_prep(gru_wih_1f, gru_whh_1f,
                                     gru_bih_1f, gru_bhh_1f)
    w1b, u1b, gb1b, bn1b = gate_prep(gru_wih_1b, gru_whh_1b,
                                     gru_bih_1b, gru_bhh_1b)

    gi0f = _mm_bias(xs_flat, w0f, gb0f, 512)
    gi0b = _mm_bias(xs_flat, w0b, gb0b, 512)
    y0 = _scan_mid(gi0f, gi0b, u0f, u0b, bn0f, bn0b)
    gi1f = _mm_bias(y0, w1f, gb1f, 512)
    gi1b = _mm_bias(y0, w1b, gb1b, 512)
    sout = _scan_final(gi1f, gi1b, u1f, u1b, bn1f, bn1b,
                       smlp_w1, smlp_b1.reshape(1, HH),
                       smlp_w2, smlp_b2.reshape(1, C))

    h = _proj(x, proj_w, proj_b.reshape(1, H))
    for l in range(NL):
        h_init = h
        ew = jax.nn.softmax(edge_imp[l]).reshape(1, 4)
        hp_halves = []
        for t in range(4):
            g0, g1 = _mm_scale(h, edge_w[l, t], ideg[t])
            hn0 = hn1 = None
            for _k in range(K):
                a0, a1 = _sc_scatter_pass(src2d[t], dst2d[t], g0, g1,
                                          zeros_init)
                hn0, hn1, g0, g1 = _blend(a0[:N], a1[:N], g0, g1,
                                          h_init, ideg[t])
            hp_halves.append((hn0, hn1))
        h = _epilogue(h_init, ew, ln_g[l].reshape(1, H),
                      ln_b[l].reshape(1, H), hp_halves)

    offsets = jnp.searchsorted(batch.astype(jnp.int32),
                               jnp.arange(B + 1, dtype=jnp.int32)
                               ).astype(jnp.int32)
    return _pool(offsets, h, gmlp_w1, gmlp_b1.reshape(1, HH),
                 gmlp_w2, gmlp_b2.reshape(1, C), sout)


# speedup vs baseline: 1.1461x; 1.1461x over previous
import jax
import jax.numpy as jnp
from jax import lax
from jax.experimental import pallas as pl
from jax.experimental.pallas import tpu as pltpu
from jax.experimental.pallas import tpu_sc as plsc

N, D, H, C, T, B, E, HS, K, NL = 10000, 768, 256, 14, 128, 16, 160000, 128, 8, 3
ALPHA = 0.1
HH = H // 2
NSUB = 16
WIN = 64
NBUF = 4
EPS = 10240
EPAD = EPS * NSUB
NWIN_SUB = EPS // WIN
CHUNK = 40
NCHUNK = NWIN_SUB // CHUNK
NACC = 10112
ROWS_PER_SUB = NACC // NSUB
TBF = T * B
G3 = 3 * HS


def _sc_scatter_body(src_hbm, dst_hbm, g0_hbm, g1_hbm, zeros_hbm,
                     acc0_hbm, acc1_hbm,
                     srcw, dstw, rows0, rows1, rows2, rows3, acc_sh,
                     gsem0, gsem1, gsem2, gsem3,
                     ssem0, ssem1, ssem2, ssem3):
    core = lax.axis_index("c")
    sub = lax.axis_index("s")
    zbase = sub * ROWS_PER_SUB
    pltpu.sync_copy(zeros_hbm, acc_sh.at[pl.ds(zbase, ROWS_PER_SUB)])
    plsc.subcore_barrier()

    def do_half(g_hbm, acc_out):
        base_row = sub * NWIN_SUB
        bufs = (rows0, rows1, rows2, rows3)
        gsems = (gsem0, gsem1, gsem2, gsem3)
        ssems = (ssem0, ssem1, ssem2, ssem3)

        def fire(w, b):
            pltpu.async_copy(g_hbm.at[srcw.at[w]], bufs[b], gsems[b])

        def wait_gather(w, b):
            pltpu.make_async_copy(g_hbm.at[srcw.at[w]], bufs[b],
                                  gsems[b]).wait()

        def scatter(w, b):
            pltpu.async_copy(bufs[b], acc_sh.at[dstw.at[w]], ssems[b],
                             add=True)

        def wait_scatter(w, b):
            pltpu.make_async_copy(bufs[b], acc_sh.at[dstw.at[w]],
                                  ssems[b]).wait()

        @pl.loop(0, NCHUNK)
        def _(c):
            crow = base_row + c * CHUNK
            pltpu.sync_copy(src_hbm.at[pl.ds(crow, CHUNK)], srcw)
            pltpu.sync_copy(dst_hbm.at[pl.ds(crow, CHUNK)], dstw)
            fire(0, 0)
            fire(1, 1)
            for w in (0, 1):
                wait_gather(w, w)
                scatter(w, w)
                fire(w + 2, w + 2)

            @pl.loop(0, (CHUNK - 4) // NBUF)
            def _(i):
                for j in range(NBUF):
                    w = 2 + NBUF * i + j
                    b = (2 + j) % NBUF
                    b2 = j % NBUF
                    wait_gather(w, b)
                    scatter(w, b)
                    wait_scatter(w - 2, b2)
                    fire(w + 2, b2)

            for j in range(2):
                w = CHUNK - 2 + j
                b = (2 + j) % NBUF
                wait_gather(w, b)
                scatter(w, b)
            for j in range(NBUF):
                w = CHUNK - NBUF + j
                wait_scatter(w, w % NBUF)

        plsc.subcore_barrier()
        obase = sub * ROWS_PER_SUB
        pltpu.sync_copy(acc_sh.at[pl.ds(obase, ROWS_PER_SUB)],
                        acc_out.at[pl.ds(obase, ROWS_PER_SUB)])

    @pl.when(core == 0)
    def _():
        do_half(g0_hbm, acc0_hbm)

    @pl.when(core == 1)
    def _():
        do_half(g1_hbm, acc1_hbm)


def _sc_scatter_pass(src2d, dst2d, g0, g1, zeros_init):
    mesh = plsc.VectorSubcoreMesh(core_axis_name="c", subcore_axis_name="s")
    acc_ty = jax.ShapeDtypeStruct((NACC, HH), jnp.float32)
    kern = pl.kernel(
        _sc_scatter_body,
        out_type=[acc_ty, acc_ty],
        mesh=mesh,
        scratch_types=[
            pltpu.VMEM((CHUNK, WIN), jnp.int32),
            pltpu.VMEM((CHUNK, WIN), jnp.int32),
            pltpu.VMEM((WIN, HH), jnp.float32),
            pltpu.VMEM((WIN, HH), jnp.float32),
            pltpu.VMEM((WIN, HH), jnp.float32),
            pltpu.VMEM((WIN, HH), jnp.float32),
            pltpu.VMEM_SHARED((NACC, HH), jnp.float32),
            pltpu.SemaphoreType.DMA,
            pltpu.SemaphoreType.DMA,
            pltpu.SemaphoreType.DMA,
            pltpu.SemaphoreType.DMA,
            pltpu.SemaphoreType.DMA,
            pltpu.SemaphoreType.DMA,
            pltpu.SemaphoreType.DMA,
            pltpu.SemaphoreType.DMA,
        ],
    )
    return kern(src2d, dst2d, g0, g1, zeros_init)


MBLK = 1000


def _proj_body(x_ref, w_ref, b_ref, o_ref):
    o_ref[...] = jnp.maximum(
        jnp.dot(x_ref[...], w_ref[...], preferred_element_type=jnp.float32)
        + b_ref[...], 0.0)


def _proj(x, w, b):
    return pl.pallas_call(
        _proj_body,
        grid=(N // MBLK,),
        in_specs=[
            pl.BlockSpec((MBLK, D), lambda i: (i, 0)),
            pl.BlockSpec((D, H), lambda i: (0, 0)),
            pl.BlockSpec((1, H), lambda i: (0, 0)),
        ],
        out_specs=pl.BlockSpec((MBLK, H), lambda i: (i, 0)),
        out_shape=jax.ShapeDtypeStruct((N, H), jnp.float32),
    )(x, w, b)


def _mm_scale_body(h_ref, w_ref, ideg_ref, g0_ref, g1_ref):
    ht = jnp.dot(h_ref[...], w_ref[...], preferred_element_type=jnp.float32)
    dinv = lax.rsqrt(ideg_ref[...] + 1.0)
    g = ht * dinv
    g0_ref[...] = g[:, :HH]
    g1_ref[...] = g[:, HH:]


def _mm_scale(h, w, ideg):
    half = jax.ShapeDtypeStruct((N, HH), jnp.float32)
    return pl.pallas_call(
        _mm_scale_body,
        grid=(N // MBLK,),
        in_specs=[
            pl.BlockSpec((MBLK, H), lambda i: (i, 0)),
            pl.BlockSpec((H, H), lambda i: (0, 0)),
            pl.BlockSpec((MBLK, 1), lambda i: (i, 0)),
        ],
        out_specs=[pl.BlockSpec((MBLK, HH), lambda i: (i, 0))] * 2,
        out_shape=[half, half],
    )(h, w, ideg)


def _blend_body(a0_ref, a1_ref, g0_ref, g1_ref, hi_ref, ideg_ref,
                hn0_ref, hn1_ref, gn0_ref, gn1_ref):
    dinv = lax.rsqrt(ideg_ref[...] + 1.0)
    sc = (1.0 - ALPHA) * dinv
    h0 = (a0_ref[...] + g0_ref[...]) * sc + ALPHA * hi_ref[:, :HH]
    h1 = (a1_ref[...] + g1_ref[...]) * sc + ALPHA * hi_ref[:, HH:]
    hn0_ref[...] = h0
    hn1_ref[...] = h1
    gn0_ref[...] = h0 * dinv
    gn1_ref[...] = h1 * dinv


def _blend(a0, a1, g0, g1, h_init, ideg):
    half = jax.ShapeDtypeStruct((N, HH), jnp.float32)
    return pl.pallas_call(
        _blend_body,
        grid=(N // MBLK,),
        in_specs=[pl.BlockSpec((MBLK, HH), lambda i: (i, 0))] * 4 + [
            pl.BlockSpec((MBLK, H), lambda i: (i, 0)),
            pl.BlockSpec((MBLK, 1), lambda i: (i, 0)),
        ],
        out_specs=[pl.BlockSpec((MBLK, HH), lambda i: (i, 0))] * 4,
        out_shape=[half, half, half, half],
    )(a0, a1, g0, g1, h_init, ideg)


def _epilogue_body(h_ref, ew_ref, lng_ref, lnb_ref,
                   p00, p01, p10, p11, p20, p21, p30, p31, out_ref):
    ew = ew_ref[...]
    halves = ((p00, p01), (p10, p11), (p20, p21), (p30, p31))
    acc = jnp.zeros((MBLK, H), jnp.float32)
    for t in range(4):
        hp = jnp.concatenate([halves[t][0][...], halves[t][1][...]], axis=1)
        acc = acc + ew[0:1, t:t + 1] * hp
    m = jnp.mean(acc, axis=1, keepdims=True)
    v = jnp.mean((acc - m) ** 2, axis=1, keepdims=True)
    y = (acc - m) * lax.rsqrt(v + 1e-5) * lng_ref[...] + lnb_ref[...]
    g = y * 0.5 * (1.0 + lax.erf(y * jnp.float32(0.7071067811865476)))
    out_ref[...] = g + h_ref[...]


def _epilogue(h, ew, lng, lnb, hp_halves):
    flat = [a for pair in hp_halves for a in pair]
    return pl.pallas_call(
        _epilogue_body,
        grid=(N // MBLK,),
        in_specs=[
            pl.BlockSpec((MBLK, H), lambda i: (i, 0)),
            pl.BlockSpec((1, 4), lambda i: (0, 0)),
            pl.BlockSpec((1, H), lambda i: (0, 0)),
            pl.BlockSpec((1, H), lambda i: (0, 0)),
        ] + [pl.BlockSpec((MBLK, HH), lambda i: (i, 0))] * 8,
        out_specs=pl.BlockSpec((MBLK, H), lambda i: (i, 0)),
        out_shape=jax.ShapeDtypeStruct((N, H), jnp.float32),
    )(h, ew, lng, lnb, *flat)


def _mm_bias_body(x_ref, w_ref, b_ref, o_ref):
    o_ref[...] = jnp.dot(x_ref[...], w_ref[...],
                         preferred_element_type=jnp.float32) + b_ref[...]


def _mm_bias(x, w, b, rblk):
    rows, kdim = x.shape
    ndim = w.shape[1]
    return pl.pallas_call(
        _mm_bias_body,
        grid=(rows // rblk,),
        in_specs=[
            pl.BlockSpec((rblk, kdim), lambda i: (i, 0)),
            pl.BlockSpec((kdim, ndim), lambda i: (0, 0)),
            pl.BlockSpec((1, ndim), lambda i: (0, 0)),
        ],
        out_specs=pl.BlockSpec((rblk, ndim), lambda i: (i, 0)),
        out_shape=jax.ShapeDtypeStruct((rows, ndim), jnp.float32),
    )(x, w, b)


def _gru_cell(gi, gh, bn, h):
    r = jax.nn.sigmoid(gi[:, :HS] + gh[:, :HS])
    z = jax.nn.sigmoid(gi[:, HS:2 * HS] + gh[:, HS:2 * HS])
    nn_ = jnp.tanh(gi[:, 2 * HS:] + r * (gh[:, 2 * HS:] + bn))
    return (1.0 - z) * nn_ + z * h


def _scan_mid_body(gif_ref, gib_ref, uf_ref, ub_ref, bnf_ref, bnb_ref,
                   y_ref):
    uf = uf_ref[...]
    ub = ub_ref[...]
    bnf = bnf_ref[...]
    bnb = bnb_ref[...]

    def step(t, carry):
        hf, hb = carry
        tb = T - 1 - t
        ghf = jnp.dot(hf, uf, preferred_element_type=jnp.float32)
        ghb = jnp.dot(hb, ub, preferred_element_type=jnp.float32)
        gf = gif_ref[pl.ds(t * B, B), :]
        gb = gib_ref[pl.ds(tb * B, B), :]
        hf2 = _gru_cell(gf, ghf, bnf, hf)
        hb2 = _gru_cell(gb, ghb, bnb, hb)
        y_ref[pl.ds(t * B, B), :HS] = hf2
        y_ref[pl.ds(tb * B, B), HS:] = hb2
        return (hf2, hb2)

    h0 = jnp.zeros((B, HS), jnp.float32)
    lax.fori_loop(0, T, step, (h0, h0))


def _scan_mid(gif, gib, uf, ub, bnf, bnb):
    return pl.pallas_call(
        _scan_mid_body,
        out_shape=jax.ShapeDtypeStruct((TBF, 2 * HS), jnp.float32),
    )(gif, gib, uf, ub, bnf, bnb)


def _scan_final_body(gif_ref, gib_ref, uf_ref, ub_ref, bnf_ref, bnb_ref,
                     w1_ref, b1_ref, w2_ref, b2_ref, out_ref):
    uf = uf_ref[...]
    ub = ub_ref[...]
    bnf = bnf_ref[...]
    bnb = bnb_ref[...]
    neg = jnp.float32(-jnp.inf)

    def step(t, carry):
        hf, hb, sf, sb, mf, mb = carry
        tb = T - 1 - t
        ghf = jnp.dot(hf, uf, preferred_element_type=jnp.float32)
        ghb = jnp.dot(hb, ub, preferred_element_type=jnp.float32)
        gf = gif_ref[pl.ds(t * B, B), :]
        gb = gib_ref[pl.ds(tb * B, B), :]
        hf2 = _gru_cell(gf, ghf, bnf, hf)
        hb2 = _gru_cell(gb, ghb, bnb, hb)
        return (hf2, hb2, sf + hf2, sb + hb2,
                jnp.maximum(mf, hf2), jnp.maximum(mb, hb2))

    h0 = jnp.zeros((B, HS), jnp.float32)
    m0 = jnp.full((B, HS), neg, jnp.float32)
    _, _, sf, sb, mf, mb = lax.fori_loop(0, T, step, (h0, h0, h0, h0, m0, m0))
    sp = jnp.concatenate([sf / T + mf, sb / T + mb], axis=1)
    y = jnp.maximum(jnp.dot(sp, w1_ref[...],
                            preferred_element_type=jnp.float32)
                    + b1_ref[...], 0.0)
    out_ref[...] = jnp.dot(y, w2_ref[...],
                           preferred_element_type=jnp.float32) + b2_ref[...]


def _scan_final(gif, gib, uf, ub, bnf, bnb, w1, b1, w2, b2):
    return pl.pallas_call(
        _scan_final_body,
        out_shape=jax.ShapeDtypeStruct((B, C), jnp.float32),
    )(gif, gib, uf, ub, bnf, bnb, w1, b1, w2, b2)


def _pool_body(off_ref, h_ref, w1_ref, b1_ref, w2_ref, b2_ref, sout_ref,
               out_ref):
    for b in range(B):
        lo = off_ref[b]
        hi = off_ref[b + 1]

        def step(i, carry):
            s, m = carry
            row = h_ref[pl.ds(lo + i, 1), :]
            return (s + row, jnp.maximum(m, row))

        s0 = jnp.zeros((1, H), jnp.float32)
        m0 = jnp.full((1, H), -jnp.inf, jnp.float32)
        s, m = lax.fori_loop(0, hi - lo, step, (s0, m0))
        cnt = jnp.maximum((hi - lo).astype(jnp.float32), 1.0)
        gfeat = s / cnt + m
        y = jnp.maximum(jnp.dot(gfeat, w1_ref[...],
                                preferred_element_type=jnp.float32)
                        + b1_ref[...], 0.0)
        go = jnp.dot(y, w2_ref[...],
                     preferred_element_type=jnp.float32) + b2_ref[...]
        out_ref[pl.ds(b, 1), :] = go + sout_ref[pl.ds(b, 1), :]


def _pool(offsets, h, w1, b1, w2, b2, sout):
    return pl.pallas_call(
        _pool_body,
        in_specs=[pl.BlockSpec(memory_space=pltpu.SMEM)] + [
            pl.BlockSpec((N, H), lambda: (0, 0)),
            pl.BlockSpec((H, HH), lambda: (0, 0)),
            pl.BlockSpec((1, HH), lambda: (0, 0)),
            pl.BlockSpec((HH, C), lambda: (0, 0)),
            pl.BlockSpec((1, C), lambda: (0, 0)),
            pl.BlockSpec((B, C), lambda: (0, 0)),
        ],
        out_specs=pl.BlockSpec((B, C), lambda: (0, 0)),
        out_shape=jax.ShapeDtypeStruct((B, C), jnp.float32),
    )(offsets, h, w1, b1, w2, b2, sout)


def kernel(x, edge_ast, edge_cfg, edge_dfg, edge_cdg, batch,
           sequence_features, proj_w, proj_b, edge_w, edge_imp, ln_g, ln_b,
           gru_wih_0f, gru_whh_0f, gru_bih_0f, gru_bhh_0f,
           gru_wih_0b, gru_whh_0b, gru_bih_0b, gru_bhh_0b,
           gru_wih_1f, gru_whh_1f, gru_bih_1f, gru_bhh_1f,
           gru_wih_1b, gru_whh_1b, gru_bih_1b, gru_bhh_1b,
           gmlp_w1, gmlp_b1, gmlp_w2, gmlp_b2,
           smlp_w1, smlp_b1, smlp_w2, smlp_b2):
    f32 = jnp.float32
    edges = (edge_ast, edge_cfg, edge_dfg, edge_cdg)

    npad = EPAD - E
    pad_per_sub = npad // NSUB
    dpad_row = N + (jnp.arange(pad_per_sub, dtype=jnp.int32) % (NACC - N))
    src2d, dst2d = [], []
    for e in edges:
        e = e.astype(jnp.int32)
        s = jnp.concatenate([
            e[0].reshape(NSUB, E // NSUB),
            jnp.zeros((NSUB, pad_per_sub), jnp.int32)], axis=1)
        d = jnp.concatenate([
            e[1].reshape(NSUB, E // NSUB),
            jnp.tile(dpad_row, (NSUB, 1))], axis=1)
        src2d.append(s.reshape(EPAD // WIN, WIN))
        dst2d.append(d.reshape(EPAD // WIN, WIN))

    zeros_init = jnp.zeros((ROWS_PER_SUB, HH), f32)
    ones_tab = jnp.ones((N, HH), f32)

    ideg = []
    for t in range(4):
        a0, _ = _sc_scatter_pass(src2d[t], dst2d[t], ones_tab, ones_tab,
                                 zeros_init)
        ideg.append(a0[:N, 0:1])

    xs_flat = jnp.swapaxes(sequence_features, 0, 1).reshape(TBF, D)

    def gate_prep(wih, whh, bih, bhh):
        gb = bih + jnp.concatenate([bhh[:2 * HS], jnp.zeros((HS,), f32)])
        return (wih.T, whh.T, gb.reshape(1, G3),
                bhh[2 * HS:].reshape(1, HS))

    w0f, u0f, gb0f, bn0f = gate_prep(gru_wih_0f, gru_whh_0f,
                                     gru_bih_0f, gru_bhh_0f)
    w0b, u0b, gb0b, bn0b = gate_prep(gru_wih_0b, gru_whh_0b,
                                     gru_bih_0b, gru_bhh_0b)
    w1f, u1f, gb1f, bn1f = gate_prep(gru_wih_1f, gru_whh_1f,
                                     gru_bih_1f, gru_bhh_1f)
    w1b, u1b, gb1b, bn1b = gate_prep(gru_wih_1b, gru_whh_1b,
                                     gru_bih_1b, gru_bhh_1b)

    gi0f = _mm_bias(xs_flat, w0f, gb0f, 512)
    gi0b = _mm_bias(xs_flat, w0b, gb0b, 512)
    y0 = _scan_mid(gi0f, gi0b, u0f, u0b, bn0f, bn0b)
    gi1f = _mm_bias(y0, w1f, gb1f, 512)
    gi1b = _mm_bias(y0, w1b, gb1b, 512)
    sout = _scan_final(gi1f, gi1b, u1f, u1b, bn1f, bn1b,
                       smlp_w1, smlp_b1.reshape(1, HH),
                       smlp_w2, smlp_b2.reshape(1, C))

    h = _proj(x, proj_w, proj_b.reshape(1, H))
    for l in range(NL):
        h_init = h
        ew = jax.nn.softmax(edge_imp[l]).reshape(1, 4)
        hp_halves = []
        for t in range(4):
            g0, g1 = _mm_scale(h, edge_w[l, t], ideg[t])
            hn0 = hn1 = None
            for _k in range(K):
                a0, a1 = _sc_scatter_pass(src2d[t], dst2d[t], g0, g1,
                                          zeros_init)
                hn0, hn1, g0, g1 = _blend(a0[:N], a1[:N], g0, g1,
                                          h_init, ideg[t])
            hp_halves.append((hn0, hn1))
        h = _epilogue(h_init, ew, ln_g[l].reshape(1, H),
                      ln_b[l].reshape(1, H), hp_halves)

    offsets = jnp.searchsorted(batch.astype(jnp.int32),
                               jnp.arange(B + 1, dtype=jnp.int32)
                               ).astype(jnp.int32)
    return _pool(offsets, h, gmlp_w1, gmlp_b1.reshape(1, HH),
                 gmlp_w2, gmlp_b2.reshape(1, C), sout)


# speedup vs baseline: 1.2340x; 1.0767x over previous
import jax
import jax.numpy as jnp
from jax import lax
from jax.experimental import pallas as pl
from jax.experimental.pallas import tpu as pltpu
from jax.experimental.pallas import tpu_sc as plsc

N, D, H, C, T, B, E, HS, K, NL = 10000, 768, 256, 14, 128, 16, 160000, 128, 8, 3
ALPHA = 0.1
HH = H // 2
NSUB = 16
WIN = 64
NBUF = 4
EPS = 10240
EPAD = EPS * NSUB
NWIN_SUB = EPS // WIN
CHUNK = 40
NCHUNK = NWIN_SUB // CHUNK
NACC = 10112
ROWS_PER_SUB = NACC // NSUB
TBF = T * B
G3 = 3 * HS


def _sc_scatter_body(src_hbm, dst_hbm, g0_hbm, g1_hbm, zeros_hbm,
                     acc0_hbm, acc1_hbm,
                     srcw, dstw, rows0, rows1, rows2, rows3, acc_sh,
                     gsem0, gsem1, gsem2, gsem3,
                     ssem0, ssem1, ssem2, ssem3):
    core = lax.axis_index("c")
    sub = lax.axis_index("s")
    zbase = sub * ROWS_PER_SUB
    pltpu.sync_copy(zeros_hbm, acc_sh.at[pl.ds(zbase, ROWS_PER_SUB)])
    plsc.subcore_barrier()

    def do_half(g_hbm, acc_out):
        base_row = sub * NWIN_SUB
        bufs = (rows0, rows1, rows2, rows3)
        gsems = (gsem0, gsem1, gsem2, gsem3)
        ssems = (ssem0, ssem1, ssem2, ssem3)

        def fire(w, b):
            pltpu.async_copy(g_hbm.at[srcw.at[w]], bufs[b], gsems[b])

        def wait_gather(w, b):
            pltpu.make_async_copy(g_hbm.at[srcw.at[w]], bufs[b],
                                  gsems[b]).wait()

        def scatter(w, b):
            pltpu.async_copy(bufs[b], acc_sh.at[dstw.at[w]], ssems[b],
                             add=True)

        def wait_scatter(w, b):
            pltpu.make_async_copy(bufs[b], acc_sh.at[dstw.at[w]],
                                  ssems[b]).wait()

        @pl.loop(0, NCHUNK)
        def _(c):
            crow = base_row + c * CHUNK
            pltpu.sync_copy(src_hbm.at[pl.ds(crow, CHUNK)], srcw)
            pltpu.sync_copy(dst_hbm.at[pl.ds(crow, CHUNK)], dstw)
            for b in range(NBUF):
                fire(b, b)

            @pl.loop(0, CHUNK // NBUF - 1)
            def _(i):
                for b in range(NBUF):
                    w = NBUF * i + b
                    wait_gather(w, b)
                    pltpu.sync_copy(bufs[b], acc_sh.at[dstw.at[w]],
                                    add=True)
                    fire(w + NBUF, b)

            for b in range(NBUF):
                w = CHUNK - NBUF + b
                wait_gather(w, b)
                pltpu.sync_copy(bufs[b], acc_sh.at[dstw.at[w]], add=True)

        plsc.subcore_barrier()
        obase = sub * ROWS_PER_SUB
        pltpu.sync_copy(acc_sh.at[pl.ds(obase, ROWS_PER_SUB)],
                        acc_out.at[pl.ds(obase, ROWS_PER_SUB)])

    @pl.when(core == 0)
    def _():
        do_half(g0_hbm, acc0_hbm)

    @pl.when(core == 1)
    def _():
        do_half(g1_hbm, acc1_hbm)


def _sc_scatter_pass(src2d, dst2d, g0, g1, zeros_init):
    mesh = plsc.VectorSubcoreMesh(core_axis_name="c", subcore_axis_name="s")
    acc_ty = jax.ShapeDtypeStruct((NACC, HH), jnp.float32)
    kern = pl.kernel(
        _sc_scatter_body,
        out_type=[acc_ty, acc_ty],
        mesh=mesh,
        scratch_types=[
            pltpu.VMEM((CHUNK, WIN), jnp.int32),
            pltpu.VMEM((CHUNK, WIN), jnp.int32),
            pltpu.VMEM((WIN, HH), jnp.float32),
            pltpu.VMEM((WIN, HH), jnp.float32),
            pltpu.VMEM((WIN, HH), jnp.float32),
            pltpu.VMEM((WIN, HH), jnp.float32),
            pltpu.VMEM_SHARED((NACC, HH), jnp.float32),
            pltpu.SemaphoreType.DMA,
            pltpu.SemaphoreType.DMA,
            pltpu.SemaphoreType.DMA,
            pltpu.SemaphoreType.DMA,
            pltpu.SemaphoreType.DMA,
            pltpu.SemaphoreType.DMA,
            pltpu.SemaphoreType.DMA,
            pltpu.SemaphoreType.DMA,
        ],
    )
    return kern(src2d, dst2d, g0, g1, zeros_init)


MBLK = 1000


def _proj_body(x_ref, w_ref, b_ref, o_ref):
    o_ref[...] = jnp.maximum(
        jnp.dot(x_ref[...], w_ref[...], preferred_element_type=jnp.float32)
        + b_ref[...], 0.0)


def _proj(x, w, b):
    return pl.pallas_call(
        _proj_body,
        grid=(N // MBLK,),
        in_specs=[
            pl.BlockSpec((MBLK, D), lambda i: (i, 0)),
            pl.BlockSpec((D, H), lambda i: (0, 0)),
            pl.BlockSpec((1, H), lambda i: (0, 0)),
        ],
        out_specs=pl.BlockSpec((MBLK, H), lambda i: (i, 0)),
        out_shape=jax.ShapeDtypeStruct((N, H), jnp.float32),
    )(x, w, b)


def _mm_scale_body(h_ref, w_ref, ideg_ref, g0_ref, g1_ref):
    ht = jnp.dot(h_ref[...], w_ref[...], preferred_element_type=jnp.float32)
    dinv = lax.rsqrt(ideg_ref[...] + 1.0)
    g = ht * dinv
    g0_ref[...] = g[:, :HH]
    g1_ref[...] = g[:, HH:]


def _mm_scale(h, w, ideg):
    half = jax.ShapeDtypeStruct((N, HH), jnp.float32)
    return pl.pallas_call(
        _mm_scale_body,
        grid=(N // MBLK,),
        in_specs=[
            pl.BlockSpec((MBLK, H), lambda i: (i, 0)),
            pl.BlockSpec((H, H), lambda i: (0, 0)),
            pl.BlockSpec((MBLK, 1), lambda i: (i, 0)),
        ],
        out_specs=[pl.BlockSpec((MBLK, HH), lambda i: (i, 0))] * 2,
        out_shape=[half, half],
    )(h, w, ideg)


def _blend_body(a0_ref, a1_ref, g0_ref, g1_ref, hi_ref, ideg_ref,
                hn0_ref, hn1_ref, gn0_ref, gn1_ref):
    dinv = lax.rsqrt(ideg_ref[...] + 1.0)
    sc = (1.0 - ALPHA) * dinv
    h0 = (a0_ref[...] + g0_ref[...]) * sc + ALPHA * hi_ref[:, :HH]
    h1 = (a1_ref[...] + g1_ref[...]) * sc + ALPHA * hi_ref[:, HH:]
    hn0_ref[...] = h0
    hn1_ref[...] = h1
    gn0_ref[...] = h0 * dinv
    gn1_ref[...] = h1 * dinv


def _blend(a0, a1, g0, g1, h_init, ideg):
    half = jax.ShapeDtypeStruct((N, HH), jnp.float32)
    return pl.pallas_call(
        _blend_body,
        grid=(N // MBLK,),
        in_specs=[pl.BlockSpec((MBLK, HH), lambda i: (i, 0))] * 4 + [
            pl.BlockSpec((MBLK, H), lambda i: (i, 0)),
            pl.BlockSpec((MBLK, 1), lambda i: (i, 0)),
        ],
        out_specs=[pl.BlockSpec((MBLK, HH), lambda i: (i, 0))] * 4,
        out_shape=[half, half, half, half],
    )(a0, a1, g0, g1, h_init, ideg)


def _epilogue_body(h_ref, ew_ref, lng_ref, lnb_ref,
                   p00, p01, p10, p11, p20, p21, p30, p31, out_ref):
    ew = ew_ref[...]
    halves = ((p00, p01), (p10, p11), (p20, p21), (p30, p31))
    acc = jnp.zeros((MBLK, H), jnp.float32)
    for t in range(4):
        hp = jnp.concatenate([halves[t][0][...], halves[t][1][...]], axis=1)
        acc = acc + ew[0:1, t:t + 1] * hp
    m = jnp.mean(acc, axis=1, keepdims=True)
    v = jnp.mean((acc - m) ** 2, axis=1, keepdims=True)
    y = (acc - m) * lax.rsqrt(v + 1e-5) * lng_ref[...] + lnb_ref[...]
    g = y * 0.5 * (1.0 + lax.erf(y * jnp.float32(0.7071067811865476)))
    out_ref[...] = g + h_ref[...]


def _epilogue(h, ew, lng, lnb, hp_halves):
    flat = [a for pair in hp_halves for a in pair]
    return pl.pallas_call(
        _epilogue_body,
        grid=(N // MBLK,),
        in_specs=[
            pl.BlockSpec((MBLK, H), lambda i: (i, 0)),
            pl.BlockSpec((1, 4), lambda i: (0, 0)),
            pl.BlockSpec((1, H), lambda i: (0, 0)),
            pl.BlockSpec((1, H), lambda i: (0, 0)),
        ] + [pl.BlockSpec((MBLK, HH), lambda i: (i, 0))] * 8,
        out_specs=pl.BlockSpec((MBLK, H), lambda i: (i, 0)),
        out_shape=jax.ShapeDtypeStruct((N, H), jnp.float32),
    )(h, ew, lng, lnb, *flat)


def _mm_bias_body(x_ref, w_ref, b_ref, o_ref):
    o_ref[...] = jnp.dot(x_ref[...], w_ref[...],
                         preferred_element_type=jnp.float32) + b_ref[...]


def _mm_bias(x, w, b, rblk):
    rows, kdim = x.shape
    ndim = w.shape[1]
    return pl.pallas_call(
        _mm_bias_body,
        grid=(rows // rblk,),
        in_specs=[
            pl.BlockSpec((rblk, kdim), lambda i: (i, 0)),
            pl.BlockSpec((kdim, ndim), lambda i: (0, 0)),
            pl.BlockSpec((1, ndim), lambda i: (0, 0)),
        ],
        out_specs=pl.BlockSpec((rblk, ndim), lambda i: (i, 0)),
        out_shape=jax.ShapeDtypeStruct((rows, ndim), jnp.float32),
    )(x, w, b)


def _gru_cell(gi, gh, bn, h):
    r = jax.nn.sigmoid(gi[:, :HS] + gh[:, :HS])
    z = jax.nn.sigmoid(gi[:, HS:2 * HS] + gh[:, HS:2 * HS])
    nn_ = jnp.tanh(gi[:, 2 * HS:] + r * (gh[:, 2 * HS:] + bn))
    return (1.0 - z) * nn_ + z * h


def _scan_mid_body(gif_ref, gib_ref, uf_ref, ub_ref, bnf_ref, bnb_ref,
                   y_ref):
    uf = uf_ref[...]
    ub = ub_ref[...]
    bnf = bnf_ref[...]
    bnb = bnb_ref[...]

    def step(t, carry):
        hf, hb = carry
        tb = T - 1 - t
        ghf = jnp.dot(hf, uf, preferred_element_type=jnp.float32)
        ghb = jnp.dot(hb, ub, preferred_element_type=jnp.float32)
        gf = gif_ref[pl.ds(t * B, B), :]
        gb = gib_ref[pl.ds(tb * B, B), :]
        hf2 = _gru_cell(gf, ghf, bnf, hf)
        hb2 = _gru_cell(gb, ghb, bnb, hb)
        y_ref[pl.ds(t * B, B), :HS] = hf2
        y_ref[pl.ds(tb * B, B), HS:] = hb2
        return (hf2, hb2)

    h0 = jnp.zeros((B, HS), jnp.float32)
    lax.fori_loop(0, T, step, (h0, h0))


def _scan_mid(gif, gib, uf, ub, bnf, bnb):
    return pl.pallas_call(
        _scan_mid_body,
        out_shape=jax.ShapeDtypeStruct((TBF, 2 * HS), jnp.float32),
    )(gif, gib, uf, ub, bnf, bnb)


def _scan_final_body(gif_ref, gib_ref, uf_ref, ub_ref, bnf_ref, bnb_ref,
                     w1_ref, b1_ref, w2_ref, b2_ref, out_ref):
    uf = uf_ref[...]
    ub = ub_ref[...]
    bnf = bnf_ref[...]
    bnb = bnb_ref[...]
    neg = jnp.float32(-jnp.inf)

    def step(t, carry):
        hf, hb, sf, sb, mf, mb = carry
        tb = T - 1 - t
        ghf = jnp.dot(hf, uf, preferred_element_type=jnp.float32)
        ghb = jnp.dot(hb, ub, preferred_element_type=jnp.float32)
        gf = gif_ref[pl.ds(t * B, B), :]
        gb = gib_ref[pl.ds(tb * B, B), :]
        hf2 = _gru_cell(gf, ghf, bnf, hf)
        hb2 = _gru_cell(gb, ghb, bnb, hb)
        return (hf2, hb2, sf + hf2, sb + hb2,
                jnp.maximum(mf, hf2), jnp.maximum(mb, hb2))

    h0 = jnp.zeros((B, HS), jnp.float32)
    m0 = jnp.full((B, HS), neg, jnp.float32)
    _, _, sf, sb, mf, mb = lax.fori_loop(0, T, step, (h0, h0, h0, h0, m0, m0))
    sp = jnp.concatenate([sf / T + mf, sb / T + mb], axis=1)
    y = jnp.maximum(jnp.dot(sp, w1_ref[...],
                            preferred_element_type=jnp.float32)
                    + b1_ref[...], 0.0)
    out_ref[...] = jnp.dot(y, w2_ref[...],
                           preferred_element_type=jnp.float32) + b2_ref[...]


def _scan_final(gif, gib, uf, ub, bnf, bnb, w1, b1, w2, b2):
    return pl.pallas_call(
        _scan_final_body,
        out_shape=jax.ShapeDtypeStruct((B, C), jnp.float32),
    )(gif, gib, uf, ub, bnf, bnb, w1, b1, w2, b2)


def _pool_body(off_ref, h_ref, w1_ref, b1_ref, w2_ref, b2_ref, sout_ref,
               out_ref):
    for b in range(B):
        lo = off_ref[b]
        hi = off_ref[b + 1]

        def step(i, carry):
            s, m = carry
            row = h_ref[pl.ds(lo + i, 1), :]
            return (s + row, jnp.maximum(m, row))

        s0 = jnp.zeros((1, H), jnp.float32)
        m0 = jnp.full((1, H), -jnp.inf, jnp.float32)
        s, m = lax.fori_loop(0, hi - lo, step, (s0, m0))
        cnt = jnp.maximum((hi - lo).astype(jnp.float32), 1.0)
        gfeat = s / cnt + m
        y = jnp.maximum(jnp.dot(gfeat, w1_ref[...],
                                preferred_element_type=jnp.float32)
                        + b1_ref[...], 0.0)
        go = jnp.dot(y, w2_ref[...],
                     preferred_element_type=jnp.float32) + b2_ref[...]
        out_ref[pl.ds(b, 1), :] = go + sout_ref[pl.ds(b, 1), :]


def _pool(offsets, h, w1, b1, w2, b2, sout):
    return pl.pallas_call(
        _pool_body,
        in_specs=[pl.BlockSpec(memory_space=pltpu.SMEM)] + [
            pl.BlockSpec((N, H), lambda: (0, 0)),
            pl.BlockSpec((H, HH), lambda: (0, 0)),
            pl.BlockSpec((1, HH), lambda: (0, 0)),
            pl.BlockSpec((HH, C), lambda: (0, 0)),
            pl.BlockSpec((1, C), lambda: (0, 0)),
            pl.BlockSpec((B, C), lambda: (0, 0)),
        ],
        out_specs=pl.BlockSpec((B, C), lambda: (0, 0)),
        out_shape=jax.ShapeDtypeStruct((B, C), jnp.float32),
    )(offsets, h, w1, b1, w2, b2, sout)


def kernel(x, edge_ast, edge_cfg, edge_dfg, edge_cdg, batch,
           sequence_features, proj_w, proj_b, edge_w, edge_imp, ln_g, ln_b,
           gru_wih_0f, gru_whh_0f, gru_bih_0f, gru_bhh_0f,
           gru_wih_0b, gru_whh_0b, gru_bih_0b, gru_bhh_0b,
           gru_wih_1f, gru_whh_1f, gru_bih_1f, gru_bhh_1f,
           gru_wih_1b, gru_whh_1b, gru_bih_1b, gru_bhh_1b,
           gmlp_w1, gmlp_b1, gmlp_w2, gmlp_b2,
           smlp_w1, smlp_b1, smlp_w2, smlp_b2):
    f32 = jnp.float32
    edges = (edge_ast, edge_cfg, edge_dfg, edge_cdg)

    npad = EPAD - E
    pad_per_sub = npad // NSUB
    dpad_row = N + (jnp.arange(pad_per_sub, dtype=jnp.int32) % (NACC - N))
    src2d, dst2d = [], []
    for e in edges:
        e = e.astype(jnp.int32)
        s = jnp.concatenate([
            e[0].reshape(NSUB, E // NSUB),
            jnp.zeros((NSUB, pad_per_sub), jnp.int32)], axis=1)
        d = jnp.concatenate([
            e[1].reshape(NSUB, E // NSUB),
            jnp.tile(dpad_row, (NSUB, 1))], axis=1)
        src2d.append(s.reshape(EPAD // WIN, WIN))
        dst2d.append(d.reshape(EPAD // WIN, WIN))

    zeros_init = jnp.zeros((ROWS_PER_SUB, HH), f32)
    ones_tab = jnp.ones((N, HH), f32)

    ideg = []
    for t in range(4):
        a0, _ = _sc_scatter_pass(src2d[t], dst2d[t], ones_tab, ones_tab,
                                 zeros_init)
        ideg.append(a0[:N, 0:1])

    xs_flat = jnp.swapaxes(sequence_features, 0, 1).reshape(TBF, D)

    def gate_prep(wih, whh, bih, bhh):
        gb = bih + jnp.concatenate([bhh[:2 * HS], jnp.zeros((HS,), f32)])
        return (wih.T, whh.T, gb.reshape(1, G3),
                bhh[2 * HS:].reshape(1, HS))

    w0f, u0f, gb0f, bn0f = gate_prep(gru_wih_0f, gru_whh_0f,
                                     gru_bih_0f, gru_bhh_0f)
    w0b, u0b, gb0b, bn0b = gate_prep(gru_wih_0b, gru_whh_0b,
                                     gru_bih_0b, gru_bhh_0b)
    w1f, u1f, gb1f, bn1f = gate_prep(gru_wih_1f, gru_whh_1f,
                                     gru_bih_1f, gru_bhh_1f)
    w1b, u1b, gb1b, bn1b = gate_prep(gru_wih_1b, gru_whh_1b,
                                     gru_bih_1b, gru_bhh_1b)

    gi0f = _mm_bias(xs_flat, w0f, gb0f, 512)
    gi0b = _mm_bias(xs_flat, w0b, gb0b, 512)
    y0 = _scan_mid(gi0f, gi0b, u0f, u0b, bn0f, bn0b)
    gi1f = _mm_bias(y0, w1f, gb1f, 512)
    gi1b = _mm_bias(y0, w1b, gb1b, 512)
    sout = _scan_final(gi1f, gi1b, u1f, u1b, bn1f, bn1b,
                       smlp_w1, smlp_b1.reshape(1, HH),
                       smlp_w2, smlp_b2.reshape(1, C))

    h = _proj(x, proj_w, proj_b.reshape(1, H))
    for l in range(NL):
        h_init = h
        ew = jax.nn.softmax(edge_imp[l]).reshape(1, 4)
        hp_halves = []
        for t in range(4):
            g0, g1 = _mm_scale(h, edge_w[l, t], ideg[t])
            hn0 = hn1 = None
            for _k in range(K):
                a0, a1 = _sc_scatter_pass(src2d[t], dst2d[t], g0, g1,
                                          zeros_init)
                hn0, hn1, g0, g1 = _blend(a0[:N], a1[:N], g0, g1,
                                          h_init, ideg[t])
            hp_halves.append((hn0, hn1))
        h = _epilogue(h_init, ew, ln_g[l].reshape(1, H),
                      ln_b[l].reshape(1, H), hp_halves)

    offsets = jnp.searchsorted(batch.astype(jnp.int32),
                               jnp.arange(B + 1, dtype=jnp.int32)
                               ).astype(jnp.int32)
    return _pool(offsets, h, gmlp_w1, gmlp_b1.reshape(1, HH),
                 gmlp_w2, gmlp_b2.reshape(1, C), sout)


# speedup vs baseline: 1.2342x; 1.0002x over previous
import jax
import jax.numpy as jnp
from jax import lax
from jax.experimental import pallas as pl
from jax.experimental.pallas import tpu as pltpu
from jax.experimental.pallas import tpu_sc as plsc

N, D, H, C, T, B, E, HS, K, NL = 10000, 768, 256, 14, 128, 16, 160000, 128, 8, 3
ALPHA = 0.1
HH = H // 2
NSUB = 16
WIN = 64
NBUF = 4
EPS = 10240
EPAD = EPS * NSUB
NWIN_SUB = EPS // WIN
CHUNK = 40
NCHUNK = NWIN_SUB // CHUNK
NACC = 10112
ROWS_PER_SUB = NACC // NSUB
TBF = T * B
G3 = 3 * HS


def _sc_scatter_body(src_hbm, dst_hbm, g0_hbm, g1_hbm, zeros_hbm,
                     acc0_hbm, acc1_hbm,
                     srcw, dstw, rows0, rows1, rows2, rows3, acc_sh,
                     gsem0, gsem1, gsem2, gsem3):
    core = lax.axis_index("c")
    sub = lax.axis_index("s")
    zbase = sub * ROWS_PER_SUB
    pltpu.sync_copy(zeros_hbm, acc_sh.at[pl.ds(zbase, ROWS_PER_SUB)])
    plsc.subcore_barrier()

    def do_half(g_hbm, acc_out):
        base_row = sub * NWIN_SUB
        bufs = (rows0, rows1, rows2, rows3)
        gsems = (gsem0, gsem1, gsem2, gsem3)

        def fire(w, b):
            pltpu.async_copy(g_hbm.at[srcw.at[w]], bufs[b], gsems[b])

        def wait_gather(w, b):
            pltpu.make_async_copy(g_hbm.at[srcw.at[w]], bufs[b],
                                  gsems[b]).wait()

        @pl.loop(0, NCHUNK)
        def _(c):
            crow = base_row + c * CHUNK
            pltpu.sync_copy(src_hbm.at[pl.ds(crow, CHUNK)], srcw)
            pltpu.sync_copy(dst_hbm.at[pl.ds(crow, CHUNK)], dstw)
            for b in range(NBUF):
                fire(b, b)

            @pl.loop(0, CHUNK // NBUF - 1)
            def _(i):
                for b in range(NBUF):
                    w = NBUF * i + b
                    wait_gather(w, b)
                    pltpu.sync_copy(bufs[b], acc_sh.at[dstw.at[w]],
                                    add=True)
                    fire(w + NBUF, b)

            for b in range(NBUF):
                w = CHUNK - NBUF + b
                wait_gather(w, b)
                pltpu.sync_copy(bufs[b], acc_sh.at[dstw.at[w]], add=True)

        plsc.subcore_barrier()
        obase = sub * ROWS_PER_SUB
        pltpu.sync_copy(acc_sh.at[pl.ds(obase, ROWS_PER_SUB)],
                        acc_out.at[pl.ds(obase, ROWS_PER_SUB)])

    @pl.when(core == 0)
    def _():
        do_half(g0_hbm, acc0_hbm)

    @pl.when(core == 1)
    def _():
        do_half(g1_hbm, acc1_hbm)


def _sc_scatter_pass(src2d, dst2d, g0, g1, zeros_init):
    mesh = plsc.VectorSubcoreMesh(core_axis_name="c", subcore_axis_name="s")
    acc_ty = jax.ShapeDtypeStruct((NACC, HH), jnp.float32)
    kern = pl.kernel(
        _sc_scatter_body,
        out_type=[acc_ty, acc_ty],
        mesh=mesh,
        scratch_types=[
            pltpu.VMEM((CHUNK, WIN), jnp.int32),
            pltpu.VMEM((CHUNK, WIN), jnp.int32),
            pltpu.VMEM((WIN, HH), jnp.float32),
            pltpu.VMEM((WIN, HH), jnp.float32),
            pltpu.VMEM((WIN, HH), jnp.float32),
            pltpu.VMEM((WIN, HH), jnp.float32),
            pltpu.VMEM_SHARED((NACC, HH), jnp.float32),
            pltpu.SemaphoreType.DMA,
            pltpu.SemaphoreType.DMA,
            pltpu.SemaphoreType.DMA,
            pltpu.SemaphoreType.DMA,
        ],
    )
    return kern(src2d, dst2d, g0, g1, zeros_init)


MBLK = 1000


def _proj_body(x_ref, w_ref, b_ref, o_ref):
    o_ref[...] = jnp.maximum(
        jnp.dot(x_ref[...], w_ref[...], preferred_element_type=jnp.float32)
        + b_ref[...], 0.0)


def _proj(x, w, b):
    return pl.pallas_call(
        _proj_body,
        grid=(N // MBLK,),
        in_specs=[
            pl.BlockSpec((MBLK, D), lambda i: (i, 0)),
            pl.BlockSpec((D, H), lambda i: (0, 0)),
            pl.BlockSpec((1, H), lambda i: (0, 0)),
        ],
        out_specs=pl.BlockSpec((MBLK, H), lambda i: (i, 0)),
        out_shape=jax.ShapeDtypeStruct((N, H), jnp.float32),
    )(x, w, b)


def _mm_scale_body(h_ref, w_ref, ideg_ref, g0_ref, g1_ref):
    ht = jnp.dot(h_ref[...], w_ref[...], preferred_element_type=jnp.float32)
    dinv = lax.rsqrt(ideg_ref[...] + 1.0)
    g = ht * dinv
    g0_ref[...] = g[:, :HH]
    g1_ref[...] = g[:, HH:]


def _mm_scale(h, w, ideg):
    half = jax.ShapeDtypeStruct((N, HH), jnp.float32)
    return pl.pallas_call(
        _mm_scale_body,
        grid=(N // MBLK,),
        in_specs=[
            pl.BlockSpec((MBLK, H), lambda i: (i, 0)),
            pl.BlockSpec((H, H), lambda i: (0, 0)),
            pl.BlockSpec((MBLK, 1), lambda i: (i, 0)),
        ],
        out_specs=[pl.BlockSpec((MBLK, HH), lambda i: (i, 0))] * 2,
        out_shape=[half, half],
    )(h, w, ideg)


def _blend_body(a0_ref, a1_ref, g0_ref, g1_ref, hi_ref, ideg_ref,
                hn0_ref, hn1_ref, gn0_ref, gn1_ref):
    dinv = lax.rsqrt(ideg_ref[...] + 1.0)
    sc = (1.0 - ALPHA) * dinv
    h0 = (a0_ref[...] + g0_ref[...]) * sc + ALPHA * hi_ref[:, :HH]
    h1 = (a1_ref[...] + g1_ref[...]) * sc + ALPHA * hi_ref[:, HH:]
    hn0_ref[...] = h0
    hn1_ref[...] = h1
    gn0_ref[...] = h0 * dinv
    gn1_ref[...] = h1 * dinv


def _blend(a0, a1, g0, g1, h_init, ideg):
    half = jax.ShapeDtypeStruct((N, HH), jnp.float32)
    return pl.pallas_call(
        _blend_body,
        grid=(N // MBLK,),
        in_specs=[pl.BlockSpec((MBLK, HH), lambda i: (i, 0))] * 4 + [
            pl.BlockSpec((MBLK, H), lambda i: (i, 0)),
            pl.BlockSpec((MBLK, 1), lambda i: (i, 0)),
        ],
        out_specs=[pl.BlockSpec((MBLK, HH), lambda i: (i, 0))] * 4,
        out_shape=[half, half, half, half],
    )(a0, a1, g0, g1, h_init, ideg)


def _epilogue_body(h_ref, ew_ref, lng_ref, lnb_ref,
                   p00, p01, p10, p11, p20, p21, p30, p31, out_ref):
    ew = ew_ref[...]
    halves = ((p00, p01), (p10, p11), (p20, p21), (p30, p31))
    acc = jnp.zeros((MBLK, H), jnp.float32)
    for t in range(4):
        hp = jnp.concatenate([halves[t][0][...], halves[t][1][...]], axis=1)
        acc = acc + ew[0:1, t:t + 1] * hp
    m = jnp.mean(acc, axis=1, keepdims=True)
    v = jnp.mean((acc - m) ** 2, axis=1, keepdims=True)
    y = (acc - m) * lax.rsqrt(v + 1e-5) * lng_ref[...] + lnb_ref[...]
    g = y * 0.5 * (1.0 + lax.erf(y * jnp.float32(0.7071067811865476)))
    out_ref[...] = g + h_ref[...]


def _epilogue(h, ew, lng, lnb, hp_halves):
    flat = [a for pair in hp_halves for a in pair]
    return pl.pallas_call(
        _epilogue_body,
        grid=(N // MBLK,),
        in_specs=[
            pl.BlockSpec((MBLK, H), lambda i: (i, 0)),
            pl.BlockSpec((1, 4), lambda i: (0, 0)),
            pl.BlockSpec((1, H), lambda i: (0, 0)),
            pl.BlockSpec((1, H), lambda i: (0, 0)),
        ] + [pl.BlockSpec((MBLK, HH), lambda i: (i, 0))] * 8,
        out_specs=pl.BlockSpec((MBLK, H), lambda i: (i, 0)),
        out_shape=jax.ShapeDtypeStruct((N, H), jnp.float32),
    )(h, ew, lng, lnb, *flat)


def _mm_bias_body(x_ref, w_ref, b_ref, o_ref):
    o_ref[...] = jnp.dot(x_ref[...], w_ref[...],
                         preferred_element_type=jnp.float32) + b_ref[...]


def _mm_bias(x, w, b, rblk):
    rows, kdim = x.shape
    ndim = w.shape[1]
    return pl.pallas_call(
        _mm_bias_body,
        grid=(rows // rblk,),
        in_specs=[
            pl.BlockSpec((rblk, kdim), lambda i: (i, 0)),
            pl.BlockSpec((kdim, ndim), lambda i: (0, 0)),
            pl.BlockSpec((1, ndim), lambda i: (0, 0)),
        ],
        out_specs=pl.BlockSpec((rblk, ndim), lambda i: (i, 0)),
        out_shape=jax.ShapeDtypeStruct((rows, ndim), jnp.float32),
    )(x, w, b)


def _gru_cell(gi, gh, bn, h):
    r = jax.nn.sigmoid(gi[:, :HS] + gh[:, :HS])
    z = jax.nn.sigmoid(gi[:, HS:2 * HS] + gh[:, HS:2 * HS])
    nn_ = jnp.tanh(gi[:, 2 * HS:] + r * (gh[:, 2 * HS:] + bn))
    return (1.0 - z) * nn_ + z * h


def _scan_mid_body(gif_ref, gib_ref, uf_ref, ub_ref, bnf_ref, bnb_ref,
                   y_ref):
    uf = uf_ref[...]
    ub = ub_ref[...]
    bnf = bnf_ref[...]
    bnb = bnb_ref[...]

    def step(t, carry):
        hf, hb = carry
        tb = T - 1 - t
        ghf = jnp.dot(hf, uf, preferred_element_type=jnp.float32)
        ghb = jnp.dot(hb, ub, preferred_element_type=jnp.float32)
        gf = gif_ref[pl.ds(t * B, B), :]
        gb = gib_ref[pl.ds(tb * B, B), :]
        hf2 = _gru_cell(gf, ghf, bnf, hf)
        hb2 = _gru_cell(gb, ghb, bnb, hb)
        y_ref[pl.ds(t * B, B), :HS] = hf2
        y_ref[pl.ds(tb * B, B), HS:] = hb2
        return (hf2, hb2)

    h0 = jnp.zeros((B, HS), jnp.float32)
    lax.fori_loop(0, T, step, (h0, h0))


def _scan_mid(gif, gib, uf, ub, bnf, bnb):
    return pl.pallas_call(
        _scan_mid_body,
        out_shape=jax.ShapeDtypeStruct((TBF, 2 * HS), jnp.float32),
    )(gif, gib, uf, ub, bnf, bnb)


def _scan_final_body(gif_ref, gib_ref, uf_ref, ub_ref, bnf_ref, bnb_ref,
                     w1_ref, b1_ref, w2_ref, b2_ref, out_ref):
    uf = uf_ref[...]
    ub = ub_ref[...]
    bnf = bnf_ref[...]
    bnb = bnb_ref[...]
    neg = jnp.float32(-jnp.inf)

    def step(t, carry):
        hf, hb, sf, sb, mf, mb = carry
        tb = T - 1 - t
        ghf = jnp.dot(hf, uf, preferred_element_type=jnp.float32)
        ghb = jnp.dot(hb, ub, preferred_element_type=jnp.float32)
        gf = gif_ref[pl.ds(t * B, B), :]
        gb = gib_ref[pl.ds(tb * B, B), :]
        hf2 = _gru_cell(gf, ghf, bnf, hf)
        hb2 = _gru_cell(gb, ghb, bnb, hb)
        return (hf2, hb2, sf + hf2, sb + hb2,
                jnp.maximum(mf, hf2), jnp.maximum(mb, hb2))

    h0 = jnp.zeros((B, HS), jnp.float32)
    m0 = jnp.full((B, HS), neg, jnp.float32)
    _, _, sf, sb, mf, mb = lax.fori_loop(0, T, step, (h0, h0, h0, h0, m0, m0))
    sp = jnp.concatenate([sf / T + mf, sb / T + mb], axis=1)
    y = jnp.maximum(jnp.dot(sp, w1_ref[...],
                            preferred_element_type=jnp.float32)
                    + b1_ref[...], 0.0)
    out_ref[...] = jnp.dot(y, w2_ref[...],
                           preferred_element_type=jnp.float32) + b2_ref[...]


def _scan_final(gif, gib, uf, ub, bnf, bnb, w1, b1, w2, b2):
    return pl.pallas_call(
        _scan_final_body,
        out_shape=jax.ShapeDtypeStruct((B, C), jnp.float32),
    )(gif, gib, uf, ub, bnf, bnb, w1, b1, w2, b2)


def _pool_body(off_ref, h_ref, w1_ref, b1_ref, w2_ref, b2_ref, sout_ref,
               out_ref):
    for b in range(B):
        lo = off_ref[b]
        hi = off_ref[b + 1]

        def step(i, carry):
            s, m = carry
            row = h_ref[pl.ds(lo + i, 1), :]
            return (s + row, jnp.maximum(m, row))

        s0 = jnp.zeros((1, H), jnp.float32)
        m0 = jnp.full((1, H), -jnp.inf, jnp.float32)
        s, m = lax.fori_loop(0, hi - lo, step, (s0, m0))
        cnt = jnp.maximum((hi - lo).astype(jnp.float32), 1.0)
        gfeat = s / cnt + m
        y = jnp.maximum(jnp.dot(gfeat, w1_ref[...],
                                preferred_element_type=jnp.float32)
                        + b1_ref[...], 0.0)
        go = jnp.dot(y, w2_ref[...],
                     preferred_element_type=jnp.float32) + b2_ref[...]
        out_ref[pl.ds(b, 1), :] = go + sout_ref[pl.ds(b, 1), :]


def _pool(offsets, h, w1, b1, w2, b2, sout):
    return pl.pallas_call(
        _pool_body,
        in_specs=[pl.BlockSpec(memory_space=pltpu.SMEM)] + [
            pl.BlockSpec((N, H), lambda: (0, 0)),
            pl.BlockSpec((H, HH), lambda: (0, 0)),
            pl.BlockSpec((1, HH), lambda: (0, 0)),
            pl.BlockSpec((HH, C), lambda: (0, 0)),
            pl.BlockSpec((1, C), lambda: (0, 0)),
            pl.BlockSpec((B, C), lambda: (0, 0)),
        ],
        out_specs=pl.BlockSpec((B, C), lambda: (0, 0)),
        out_shape=jax.ShapeDtypeStruct((B, C), jnp.float32),
    )(offsets, h, w1, b1, w2, b2, sout)


def kernel(x, edge_ast, edge_cfg, edge_dfg, edge_cdg, batch,
           sequence_features, proj_w, proj_b, edge_w, edge_imp, ln_g, ln_b,
           gru_wih_0f, gru_whh_0f, gru_bih_0f, gru_bhh_0f,
           gru_wih_0b, gru_whh_0b, gru_bih_0b, gru_bhh_0b,
           gru_wih_1f, gru_whh_1f, gru_bih_1f, gru_bhh_1f,
           gru_wih_1b, gru_whh_1b, gru_bih_1b, gru_bhh_1b,
           gmlp_w1, gmlp_b1, gmlp_w2, gmlp_b2,
           smlp_w1, smlp_b1, smlp_w2, smlp_b2):
    f32 = jnp.float32
    edges = (edge_ast, edge_cfg, edge_dfg, edge_cdg)

    npad = EPAD - E
    pad_per_sub = npad // NSUB
    dpad_row = N + (jnp.arange(pad_per_sub, dtype=jnp.int32) % (NACC - N))
    src2d, dst2d = [], []
    for e in edges:
        e = e.astype(jnp.int32)
        s = jnp.concatenate([
            e[0].reshape(NSUB, E // NSUB),
            jnp.zeros((NSUB, pad_per_sub), jnp.int32)], axis=1)
        d = jnp.concatenate([
            e[1].reshape(NSUB, E // NSUB),
            jnp.tile(dpad_row, (NSUB, 1))], axis=1)
        src2d.append(s.reshape(EPAD // WIN, WIN))
        dst2d.append(d.reshape(EPAD // WIN, WIN))

    zeros_init = jnp.zeros((ROWS_PER_SUB, HH), f32)
    ones_tab = jnp.ones((N, HH), f32)

    ideg = []
    for t in range(4):
        a0, _ = _sc_scatter_pass(src2d[t], dst2d[t], ones_tab, ones_tab,
                                 zeros_init)
        ideg.append(a0[:N, 0:1])

    xs_flat = jnp.swapaxes(sequence_features, 0, 1).reshape(TBF, D)

    def gate_prep(wih, whh, bih, bhh):
        gb = bih + jnp.concatenate([bhh[:2 * HS], jnp.zeros((HS,), f32)])
        return (wih.T, whh.T, gb.reshape(1, G3),
                bhh[2 * HS:].reshape(1, HS))

    w0f, u0f, gb0f, bn0f = gate_prep(gru_wih_0f, gru_whh_0f,
                                     gru_bih_0f, gru_bhh_0f)
    w0b, u0b, gb0b, bn0b = gate_prep(gru_wih_0b, gru_whh_0b,
                                     gru_bih_0b, gru_bhh_0b)
    w1f, u1f, gb1f, bn1f = gate_prep(gru_wih_1f, gru_whh_1f,
                                     gru_bih_1f, gru_bhh_1f)
    w1b, u1b, gb1b, bn1b = gate_prep(gru_wih_1b, gru_whh_1b,
                                     gru_bih_1b, gru_bhh_1b)

    gi0f = _mm_bias(xs_flat, w0f, gb0f, 512)
    gi0b = _mm_bias(xs_flat, w0b, gb0b, 512)
    y0 = _scan_mid(gi0f, gi0b, u0f, u0b, bn0f, bn0b)
    gi1f = _mm_bias(y0, w1f, gb1f, 512)
    gi1b = _mm_bias(y0, w1b, gb1b, 512)
    sout = _scan_final(gi1f, gi1b, u1f, u1b, bn1f, bn1b,
                       smlp_w1, smlp_b1.reshape(1, HH),
                       smlp_w2, smlp_b2.reshape(1, C))

    h = _proj(x, proj_w, proj_b.reshape(1, H))
    for l in range(NL):
        h_init = h
        ew = jax.nn.softmax(edge_imp[l]).reshape(1, 4)
        hp_halves = []
        for t in range(4):
            g0, g1 = _mm_scale(h, edge_w[l, t], ideg[t])
            hn0 = hn1 = None
            for _k in range(K):
                a0, a1 = _sc_scatter_pass(src2d[t], dst2d[t], g0, g1,
                                          zeros_init)
                hn0, hn1, g0, g1 = _blend(a0[:N], a1[:N], g0, g1,
                                          h_init, ideg[t])
            hp_halves.append((hn0, hn1))
        h = _epilogue(h_init, ew, ln_g[l].reshape(1, H),
                      ln_b[l].reshape(1, H), hp_halves)

    offsets = jnp.searchsorted(batch.astype(jnp.int32),
                               jnp.arange(B + 1, dtype=jnp.int32)
                               ).astype(jnp.int32)
    return _pool(offsets, h, gmlp_w1, gmlp_b1.reshape(1, HH),
                 gmlp_w2, gmlp_b2.reshape(1, C), sout)
